# baseline (speedup 1.0000x reference)
import jax
import jax.numpy as jnp
from jax.experimental import pallas as pl
from jax.experimental.pallas import tpu as pltpu

_B = 4
_HW = 1024
_N_MEM = 4
_M = 512
_KD = 64
_VD = 64
_N = 4096
_INV_CNT = 1.0 / float(_N * _KD)


def _l2n_rows(x):
    n = jnp.sqrt(jnp.sum(x * x, axis=-1, keepdims=True))
    return x / (n + 1e-12)


def _bank_body(ca_ref, sa_ref, cb_ref, sb_ref, ma_ref, mb_ref, mem_ref,
               upd_ref, kl_ref, vl_ref):
    f32 = jnp.float32
    bf16 = jnp.bfloat16
    pid = pl.program_id(0)
    iota0 = jax.lax.broadcasted_iota(jnp.int32, (_M, _N), 0)

    ca = jnp.concatenate([ca_ref[b] for b in range(_B)], axis=1)
    sa = jnp.concatenate([sa_ref[b] for b in range(_B)], axis=1)
    cb = jnp.concatenate([cb_ref[b] for b in range(_B)], axis=1)
    sb = jnp.concatenate([sb_ref[b] for b in range(_B)], axis=1)
    mra = jnp.concatenate([ma_ref[0, b:b + 1] for b in range(_B)], axis=1)
    mrb = jnp.concatenate([mb_ref[0, b:b + 1] for b in range(_B)], axis=1)
    K = mem_ref[0, 0]
    V1 = mem_ref[0, 1]
    V2 = mem_ref[0, 2]
    qa = ca * mra
    sa_ = sa * mra
    qb = cb * mrb
    sb_ = sb * mrb

    def side(q, s, K, V):
        sc = jax.lax.dot_general(K, q, (((1,), (0,)), ((), ())),
                                 preferred_element_type=f32)
        slotmax = jnp.max(sc, axis=1, keepdims=True)
        e = jnp.exp(sc - slotmax)
        colsum = jnp.sum(e, axis=1, keepdims=True)
        qs = jnp.concatenate([q, s], axis=0)
        U = jax.lax.dot_general(e, qs, (((1,), (1,)), ((), ())),
                                preferred_element_type=f32)
        tokmax = jnp.max(sc, axis=0, keepdims=True)
        g = jnp.min(jnp.where(sc >= tokmax, iota0, _M), axis=0, keepdims=True)
        oh = (iota0 == g).astype(bf16)
        Gt = jax.lax.dot_general(
            jnp.concatenate([K, V], axis=1).astype(bf16), oh,
            (((0,), (0,)), ((), ())), preferred_element_type=f32)
        gk = Gt[0:64]
        gv = Gt[64:128]
        kpart = jnp.sum((q - gk) ** 2)
        vpart = jnp.sum((s - gv) ** 2)
        return U, colsum, gk, kpart, vpart

    Ua, csa, gka, kpa, vpa = side(qa, sa_, K, V1)
    Ub, csb, gkb, kpb, vpb = side(qb, sb_, K, V2)
    inva = 1.0 / csa
    invb = 1.0 / csb
    nk = _l2n_rows(K + Ua[:, 0:64] * inva + Ub[:, 0:64] * invb)
    nv1 = _l2n_rows(V1 + Ua[:, 64:128] * inva)
    nv2 = _l2n_rows(V2 + Ub[:, 64:128] * invb)
    upd_ref[0, 0] = nk
    upd_ref[0, 1] = nv1
    upd_ref[0, 2] = nv2

    w_last = jnp.where(pid == _N_MEM - 1, f32(1.0), f32(0.0))
    extra = (jnp.sum((ca - gka) ** 2) + jnp.sum((cb - gkb) ** 2)) * w_last
    k_step = (kpa + kpb + extra) * _INV_CNT
    v_step = (vpa + vpb) * _INV_CNT

    @pl.when(pid == 0)
    def _init():
        kl_ref[:, :] = jnp.zeros((1, 1), f32)
        vl_ref[:, :] = jnp.zeros((1, 1), f32)

    kl_ref[:, :] = kl_ref[:, :] + k_step
    vl_ref[:, :] = vl_ref[:, :] + v_step


def _attend_body(ca_ref, cb_ref, ma_ref, mb_ref, u3_ref, r1_ref, r2_ref,
                 oa_ref, ob_ref):
    f32 = jnp.float32
    bf16 = jnp.bfloat16
    nk = u3_ref[0]
    nv1 = u3_ref[1]
    nv2 = u3_ref[2]
    mra = jnp.concatenate([ma_ref[b] for b in range(_B)], axis=1)
    mrb = jnp.concatenate([mb_ref[b] for b in range(_B)], axis=1)
    qa = jnp.concatenate([ca_ref[b] for b in range(_B)], axis=1) * mra
    qb = jnp.concatenate([cb_ref[b] for b in range(_B)], axis=1) * mrb

    def attend(q, W):
        sc2 = jax.lax.dot_general(nk, q, (((1,), (0,)), ((), ())),
                                  preferred_element_type=f32)
        m2 = jnp.max(sc2, axis=0, keepdims=True)
        e2 = jnp.exp(sc2 - m2)
        inv2 = 1.0 / jnp.sum(e2, axis=0, keepdims=True)
        out = jax.lax.dot_general(W.astype(bf16), e2.astype(bf16),
                                  (((0,), (0,)), ((), ())),
                                  preferred_element_type=f32)
        return out * inv2

    oa = attend(qa, jnp.concatenate([nv1, nv2, r1_ref[:]], axis=1))
    ob = attend(qb, jnp.concatenate([nv1, nv2, r2_ref[:]], axis=1))
    for b in range(_B):
        oa_ref[b] = oa[:, b * _HW:(b + 1) * _HW]
        ob_ref[b] = ob[:, b * _HW:(b + 1) * _HW]


def kernel(conts_a, stys_a, conts_b, stys_b, masks_a, masks_b, memorys):
    B, C, H, W = conts_a.shape
    ca = conts_a.reshape(B, C, H * W)
    sa = stys_a.reshape(B, _VD, H * W)
    cb = conts_b.reshape(B, C, H * W)
    sb = stys_b.reshape(B, _VD, H * W)
    mA = masks_a.reshape(B, _N_MEM, H * W)
    mB = masks_b.reshape(B, _N_MEM, H * W)
    rng = jax.random.key(1)
    r1 = jax.random.normal(jax.random.fold_in(rng, 2 * (_N_MEM - 1)),
                           (_M, _VD), dtype=jnp.float32)
    r2 = jax.random.normal(jax.random.fold_in(rng, 2 * (_N_MEM - 1) + 1),
                           (_M, _VD), dtype=jnp.float32)
    full3 = lambda i: (0, 0, 0)
    upd, kl, vl = pl.pallas_call(
        _bank_body,
        grid=(_N_MEM,),
        in_specs=[
            pl.BlockSpec((B, C, H * W), full3),
            pl.BlockSpec((B, _VD, H * W), full3),
            pl.BlockSpec((B, C, H * W), full3),
            pl.BlockSpec((B, _VD, H * W), full3),
            pl.BlockSpec((1, B, H * W), lambda i: (i, 0, 0)),
            pl.BlockSpec((1, B, H * W), lambda i: (i, 0, 0)),
            pl.BlockSpec((1, 3, _M, _KD), lambda i: (i, 0, 0, 0)),
        ],
        out_specs=[
            pl.BlockSpec((1, 3, _M, _KD), lambda i: (i, 0, 0, 0)),
            pl.BlockSpec((1, 1), lambda i: (0, 0)),
            pl.BlockSpec((1, 1), lambda i: (0, 0)),
        ],
        compiler_params=pltpu.CompilerParams(
            vmem_limit_bytes=100 * 1024 * 1024),
        out_shape=[
            jax.ShapeDtypeStruct((_N_MEM, 3, _M, _KD), jnp.float32),
            jax.ShapeDtypeStruct((1, 1), jnp.float32),
            jax.ShapeDtypeStruct((1, 1), jnp.float32),
        ],
    )(ca, sa, cb, sb, mA.transpose(1, 0, 2), mB.transpose(1, 0, 2), memorys)

    oa, ob = pl.pallas_call(
        _attend_body,
        compiler_params=pltpu.CompilerParams(
            vmem_limit_bytes=100 * 1024 * 1024),
        out_shape=[
            jax.ShapeDtypeStruct((B, 192, H * W), jnp.float32),
            jax.ShapeDtypeStruct((B, 192, H * W), jnp.float32),
        ],
    )(ca, cb, mA[:, _N_MEM - 1:_N_MEM, :], mB[:, _N_MEM - 1:_N_MEM, :],
      upd[_N_MEM - 1], r1, r2)

    def img(x3, lo):
        return x3[:, lo:lo + _VD, :].reshape(B, _VD, H, W)

    sty_aa = img(oa, 0)
    sty_ab = img(oa, 64)
    rand_a = img(oa, 128)
    sty_ba = img(ob, 0)
    sty_bb = img(ob, 64)
    rand_b = img(ob, 128)
    return (upd, sty_aa, sty_ab, sty_ba, sty_bb,
            rand_a, rand_a, rand_b, rand_b, kl[0, 0], vl[0, 0])


# speedup vs baseline: 1.1655x; 1.1655x over previous
import jax
import jax.numpy as jnp
from jax.experimental import pallas as pl
from jax.experimental.pallas import tpu as pltpu

_B = 4
_HW = 1024
_N_MEM = 4
_M = 512
_KD = 64
_VD = 64
_N = 4096
_INV_CNT = 1.0 / float(_N * _KD)


def _l2n_rows(x):
    n = jnp.sqrt(jnp.sum(x * x, axis=-1, keepdims=True))
    return x / (n + 1e-12)


def _body(ca_ref, sa_ref, cb_ref, sb_ref, ma_ref, mb_ref, mem_ref,
          r1_ref, r2_ref, upd_ref, oaa_ref, oab_ref, ora_ref,
          oba_ref, obb_ref, orb_ref, kl_ref, vl_ref):
    f32 = jnp.float32
    bf16 = jnp.bfloat16
    iota0 = jax.lax.broadcasted_iota(jnp.int32, (_M, _N), 0)

    ca = jnp.concatenate([ca_ref[b] for b in range(_B)], axis=1)
    sa = jnp.concatenate([sa_ref[b] for b in range(_B)], axis=1)
    cb = jnp.concatenate([cb_ref[b] for b in range(_B)], axis=1)
    sb = jnp.concatenate([sb_ref[b] for b in range(_B)], axis=1)
    mas = [ma_ref[b] for b in range(_B)]
    mbs = [mb_ref[b] for b in range(_B)]
    k_sum = f32(0.0)
    v_sum = f32(0.0)

    def side(q, s, K, V, last):
        sc = jax.lax.dot_general(K, q, (((1,), (0,)), ((), ())),
                                 preferred_element_type=f32)
        slotmax = jnp.max(sc, axis=1, keepdims=True)
        e = jnp.exp(sc - slotmax)
        colsum = jnp.sum(e, axis=1, keepdims=True)
        qs = jnp.concatenate([q, s], axis=0)
        U = jax.lax.dot_general(e, qs, (((1,), (1,)), ((), ())),
                                preferred_element_type=f32)
        tokmax = jnp.max(sc, axis=0, keepdims=True)
        g = jnp.min(jnp.where(sc >= tokmax, iota0, _M), axis=0, keepdims=True)
        oh = (iota0 == g).astype(bf16)
        if last:
            Gt = jax.lax.dot_general(
                jnp.concatenate([K, V], axis=1).astype(bf16), oh,
                (((0,), (0,)), ((), ())), preferred_element_type=f32)
            gk = Gt[0:64]
            gv = Gt[64:128]
            kpart = jnp.sum((q - gk) ** 2)
        else:
            ksq8 = jnp.broadcast_to(jnp.sum(K * K, axis=1, keepdims=True),
                                    (_M, 8))
            Gt = jax.lax.dot_general(
                jnp.concatenate([V, ksq8], axis=1).astype(bf16), oh,
                (((0,), (0,)), ((), ())), preferred_element_type=f32)
            gk = None
            gv = Gt[0:64]
            kpart = (jnp.sum(q * q) - 2.0 * jnp.sum(tokmax)
                     + jnp.sum(Gt[64:72]) * 0.125)
        vpart = jnp.sum((s - gv) ** 2)
        return U, colsum, gk, kpart, vpart

    for i in range(_N_MEM):
        K = mem_ref[i, 0]
        V1 = mem_ref[i, 1]
        V2 = mem_ref[i, 2]
        mra = jnp.concatenate([mas[b][i:i + 1] for b in range(_B)], axis=1)
        mrb = jnp.concatenate([mbs[b][i:i + 1] for b in range(_B)], axis=1)
        qa = ca * mra
        sa_ = sa * mra
        qb = cb * mrb
        sb_ = sb * mrb
        last = i == _N_MEM - 1
        Ua, csa, gka, kpa, vpa = side(qa, sa_, K, V1, last)
        Ub, csb, gkb, kpb, vpb = side(qb, sb_, K, V2, last)
        inva = 1.0 / csa
        invb = 1.0 / csb
        nk = _l2n_rows(K + Ua[:, 0:64] * inva + Ub[:, 0:64] * invb)
        nv1 = _l2n_rows(V1 + Ua[:, 64:128] * inva)
        nv2 = _l2n_rows(V2 + Ub[:, 64:128] * invb)
        upd_ref[i, 0] = nk
        upd_ref[i, 1] = nv1
        upd_ref[i, 2] = nv2
        k_sum = k_sum + kpa + kpb
        v_sum = v_sum + vpa + vpb
        if last:
            k_sum = k_sum + jnp.sum((ca - gka) ** 2) + jnp.sum((cb - gkb) ** 2)

            def attend(q, W):
                sc2 = jax.lax.dot_general(nk, q, (((1,), (0,)), ((), ())),
                                          preferred_element_type=f32)
                m2 = jnp.max(sc2, axis=0, keepdims=True)
                e2 = jnp.exp(sc2 - m2)
                inv2 = 1.0 / jnp.sum(e2, axis=0, keepdims=True)
                out = jax.lax.dot_general(W.astype(bf16), e2.astype(bf16),
                                          (((0,), (0,)), ((), ())),
                                          preferred_element_type=f32)
                return out * inv2

            oa = attend(qa, jnp.concatenate([nv1, nv2, r1_ref[:]], axis=1))
            ob = attend(qb, jnp.concatenate([nv1, nv2, r2_ref[:]], axis=1))
            for b in range(_B):
                lo = b * _HW
                hi = lo + _HW
                oaa_ref[b] = oa[0:64, lo:hi]
                oab_ref[b] = oa[64:128, lo:hi]
                ora_ref[b] = oa[128:192, lo:hi]
                oba_ref[b] = ob[0:64, lo:hi]
                obb_ref[b] = ob[64:128, lo:hi]
                orb_ref[b] = ob[128:192, lo:hi]
    kl_ref[:, :] = (k_sum * _INV_CNT).reshape(1, 1)
    vl_ref[:, :] = (v_sum * _INV_CNT).reshape(1, 1)


def kernel(conts_a, stys_a, conts_b, stys_b, masks_a, masks_b, memorys):
    B, C, H, W = conts_a.shape
    ca = conts_a.reshape(B, C, H * W)
    sa = stys_a.reshape(B, _VD, H * W)
    cb = conts_b.reshape(B, C, H * W)
    sb = stys_b.reshape(B, _VD, H * W)
    mA = masks_a.reshape(B, _N_MEM, H * W)
    mB = masks_b.reshape(B, _N_MEM, H * W)
    rng = jax.random.key(1)
    r1 = jax.random.normal(jax.random.fold_in(rng, 2 * (_N_MEM - 1)),
                           (_M, _VD), dtype=jnp.float32)
    r2 = jax.random.normal(jax.random.fold_in(rng, 2 * (_N_MEM - 1) + 1),
                           (_M, _VD), dtype=jnp.float32)
    imgshape = jax.ShapeDtypeStruct((B, _VD, H * W), jnp.float32)
    upd, oaa, oab, ora, oba, obb, orb, kl, vl = pl.pallas_call(
        _body,
        compiler_params=pltpu.CompilerParams(
            vmem_limit_bytes=100 * 1024 * 1024),
        out_shape=[
            jax.ShapeDtypeStruct((_N_MEM, 3, _M, _KD), jnp.float32),
            imgshape, imgshape, imgshape, imgshape, imgshape, imgshape,
            jax.ShapeDtypeStruct((1, 1), jnp.float32),
            jax.ShapeDtypeStruct((1, 1), jnp.float32),
        ],
    )(ca, sa, cb, sb, mA, mB, memorys, r1, r2)

    def img(x3):
        return x3.reshape(B, _VD, H, W)

    return (upd, img(oaa), img(oab), img(oba), img(obb),
            img(ora), img(ora), img(orb), img(orb), kl[0, 0], vl[0, 0])
